# baseline (speedup 1.0000x reference)
import jax
import jax.numpy as jnp
from jax.experimental import pallas as pl
from jax.experimental.pallas import tpu as pltpu

_BUFF = 16
_PPF = 80
_PATCH2 = 9
_C = 128
_H = 128
_W = 128
_DS = 4
_FLS_H = 512.0
_FLS_W = 512.0
_R_MIN = 0.5
_R_MAX = 30.0
_FOV_H = 130.0
_PK = _C * _PATCH2

_CB = 32
_NC = _C // _CB
_CC = 8


def _big_body(scal_ref, fmap_vm, imap_vm, f1b_ref, ib_ref, f2b_ref,
              f1o_ref, f2o_ref, io_ref):
    c = pl.program_id(0)
    r = pl.program_id(1)
    li = scal_ref[0]

    @pl.when(r == li)
    def _():
        c0 = c * _CB
        x = fmap_vm[0, pl.ds(c0, _CB)]
        f1o_ref[0] = x
        io_ref[0] = imap_vm[0, pl.ds(c0, _CB)]

        def _pool(ci, carry):
            s0 = ci * _CC
            xs = fmap_vm[0, pl.ds(c0 + s0, _CC)]
            a = xs.reshape(_CC, _H // _DS, _DS, _W).sum(axis=2)
            b = a.reshape(_CC, _H // _DS, _W // _DS, _DS).sum(axis=3)
            f2o_ref[0, pl.ds(s0, _CC)] = b * (1.0 / (_DS * _DS))
            return carry

        jax.lax.fori_loop(0, _CB // _CC, _pool, 0)

    @pl.when(r != li)
    def _():
        f1o_ref[0] = f1b_ref[0]
        io_ref[0] = ib_ref[0]
        f2o_ref[0] = f2b_ref[0]


def _small_body(scal_ref, ts_ref, coords_ref, patches_ref, pb_ref, psb_ref,
                tb_ref, sfb_ref, po_ref, pso_ref, to_ref, sfo_ref):
    r = pl.program_id(0)
    li = scal_ref[0]
    fn = scal_ref[1]

    @pl.when(r == li)
    def _():
        po_ref[0] = patches_ref[0]
        xy = coords_ref[0]
        rp = xy[1:2, :] * ((_R_MAX - _R_MIN) / _FLS_H) + _R_MIN
        th = (xy[0:1, :] * (1.0 / _FLS_W) - 0.5) * (_FOV_H * jnp.pi / 180.0)
        pso_ref[0] = jnp.concatenate(
            [rp, th, jnp.zeros((1, _PPF), jnp.float32)], axis=0)
        sfo_ref[0] = jnp.full((1, _PPF), fn, dtype=jnp.int32)

    @pl.when(r != li)
    def _():
        po_ref[0] = pb_ref[0]
        pso_ref[0] = psb_ref[0]
        sfo_ref[0] = sfb_ref[0]

    @pl.when(r == 0)
    def _():
        lanes = jax.lax.broadcasted_iota(jnp.int32, (1, _BUFF), 1)
        to_ref[...] = jnp.where(lanes == li, ts_ref[0, 0], tb_ref[...])


def kernel(fmap, imap, patches, coords, time_stamp, frame_n,
           fmap1_buf, fmap2_buf, imap_buf, patches_buf,
           patch_state_buf, time_buf, source_frame_buf):
    frame_n = jnp.asarray(frame_n, jnp.int32)
    li = frame_n % _BUFF
    scal = jnp.stack([li, frame_n])

    f32 = jnp.float32
    vmem_whole = pl.BlockSpec(memory_space=pltpu.VMEM)
    big = pl.pallas_call(
        _big_body,
        grid_spec=pltpu.PrefetchScalarGridSpec(
            num_scalar_prefetch=1,
            grid=(_NC, _BUFF),
            in_specs=[
                vmem_whole,
                vmem_whole,
                pl.BlockSpec((1, _CB, _H, _W), lambda c, r, s: (r, c, 0, 0)),
                pl.BlockSpec((1, _CB, _H, _W), lambda c, r, s: (r, c, 0, 0)),
                pl.BlockSpec((1, _CB, _H // _DS, _W // _DS),
                             lambda c, r, s: (r, c, 0, 0)),
            ],
            out_specs=[
                pl.BlockSpec((1, _CB, _H, _W), lambda c, r, s: (r, c, 0, 0)),
                pl.BlockSpec((1, _CB, _H // _DS, _W // _DS),
                             lambda c, r, s: (r, c, 0, 0)),
                pl.BlockSpec((1, _CB, _H, _W), lambda c, r, s: (r, c, 0, 0)),
            ],
        ),
        out_shape=[
            jax.ShapeDtypeStruct((_BUFF, _C, _H, _W), f32),
            jax.ShapeDtypeStruct((_BUFF, _C, _H // _DS, _W // _DS), f32),
            jax.ShapeDtypeStruct((_BUFF, _C, _H, _W), f32),
        ],
        compiler_params=pltpu.CompilerParams(
            dimension_semantics=("parallel", "parallel")),
    )
    fmap1_new, fmap2_new, imap_new = big(scal, fmap, imap,
                                         fmap1_buf, imap_buf, fmap2_buf)

    pflat = patches.reshape(1, _PPF, _PK)
    pbflat = patches_buf.reshape(_BUFF, _PPF, _PK)
    coords2 = coords[0].T.reshape(1, 2, _PPF)
    ts2 = time_stamp.reshape(1, 1)
    ps3 = jnp.swapaxes(patch_state_buf, 1, 2)
    tb2 = time_buf.reshape(1, _BUFF)
    sf3 = source_frame_buf.reshape(_BUFF, 1, _PPF)

    small = pl.pallas_call(
        _small_body,
        grid_spec=pltpu.PrefetchScalarGridSpec(
            num_scalar_prefetch=1,
            grid=(_BUFF,),
            in_specs=[
                pl.BlockSpec((1, 1), lambda r, s: (0, 0)),
                pl.BlockSpec((1, 2, _PPF), lambda r, s: (0, 0, 0)),
                pl.BlockSpec((1, _PPF, _PK), lambda r, s: (0, 0, 0)),
                pl.BlockSpec((1, _PPF, _PK), lambda r, s: (r, 0, 0)),
                pl.BlockSpec((1, 3, _PPF), lambda r, s: (r, 0, 0)),
                pl.BlockSpec((1, _BUFF), lambda r, s: (0, 0)),
                pl.BlockSpec((1, 1, _PPF), lambda r, s: (r, 0, 0)),
            ],
            out_specs=[
                pl.BlockSpec((1, _PPF, _PK), lambda r, s: (r, 0, 0)),
                pl.BlockSpec((1, 3, _PPF), lambda r, s: (r, 0, 0)),
                pl.BlockSpec((1, _BUFF), lambda r, s: (0, 0)),
                pl.BlockSpec((1, 1, _PPF), lambda r, s: (r, 0, 0)),
            ],
        ),
        out_shape=[
            jax.ShapeDtypeStruct((_BUFF, _PPF, _PK), f32),
            jax.ShapeDtypeStruct((_BUFF, 3, _PPF), f32),
            jax.ShapeDtypeStruct((1, _BUFF), f32),
            jax.ShapeDtypeStruct((_BUFF, 1, _PPF), jnp.int32),
        ],
    )
    pnew, psnew, tnew, sfnew = small(scal, ts2, coords2, pflat, pbflat,
                                     ps3, tb2, sf3)

    return (fmap1_new, fmap2_new, imap_new,
            pnew.reshape(_BUFF, _PPF, _C, _PATCH2),
            jnp.swapaxes(psnew, 1, 2),
            tnew.reshape(_BUFF),
            sfnew.reshape(_BUFF, _PPF))


# speedup vs baseline: 1.1640x; 1.1640x over previous
import jax
import jax.numpy as jnp
from jax.experimental import pallas as pl
from jax.experimental.pallas import tpu as pltpu

_BUFF = 16
_PPF = 80
_PATCH2 = 9
_C = 128
_H = 128
_W = 128
_DS = 4
_FLS_H = 512.0
_FLS_W = 512.0
_R_MIN = 0.5
_R_MAX = 30.0
_FOV_H = 130.0
_PK = _C * _PATCH2
_PW = _H // _DS

_CH = 32
_CPR = _C // _CH
_NF1 = _BUFF * _CPR
_NTOT = 2 * _NF1
_NBUF = 12
_LAG = 6


def _stream_body(scal_ref, f4, i4, f1b4, ib4, f1o4, io4, pooled_ref,
                 stage, sem_in, sem_out):
    li = scal_ref[0]

    def _in_copy(j, slot, op):
        @pl.when(j < _NF1)
        def _():
            row = j // _CPR

            @pl.when(row == li)
            def _():
                cp = pltpu.make_async_copy(f4.at[j - _CPR * li],
                                           stage.at[slot], sem_in.at[slot])
                cp.start() if op == "start" else cp.wait()

            @pl.when(row != li)
            def _():
                cp = pltpu.make_async_copy(f1b4.at[j], stage.at[slot],
                                           sem_in.at[slot])
                cp.start() if op == "start" else cp.wait()

        @pl.when(j >= _NF1)
        def _():
            jj = j - _NF1
            row = jj // _CPR

            @pl.when(row == li)
            def _():
                cp = pltpu.make_async_copy(i4.at[jj - _CPR * li],
                                           stage.at[slot], sem_in.at[slot])
                cp.start() if op == "start" else cp.wait()

            @pl.when(row != li)
            def _():
                cp = pltpu.make_async_copy(ib4.at[jj], stage.at[slot],
                                           sem_in.at[slot])
                cp.start() if op == "start" else cp.wait()

    def _out_copy(j, slot, op):
        @pl.when(j < _NF1)
        def _():
            cp = pltpu.make_async_copy(stage.at[slot], f1o4.at[j],
                                       sem_out.at[slot])
            cp.start() if op == "start" else cp.wait()

        @pl.when(j >= _NF1)
        def _():
            cp = pltpu.make_async_copy(stage.at[slot], io4.at[j - _NF1],
                                       sem_out.at[slot])
            cp.start() if op == "start" else cp.wait()

    def _maybe_pool(j, slot):
        @pl.when((j < _NF1) & (j // _CPR == li))
        def _():
            k = j - _CPR * li

            def _pool(ci, carry):
                s0 = ci * 8
                xs = stage[slot, pl.ds(s0, 8)]
                a = xs.reshape(8, _PW, _DS, _W).sum(axis=2)
                b = a.reshape(8, _PW, _PW, _DS).sum(axis=3)
                pooled_ref[pl.ds(k * _CH + s0, 8)] = b * (1.0 / (_DS * _DS))
                return carry

            jax.lax.fori_loop(0, _CH // 8, _pool, 0)

    def _loop(i, carry):
        @pl.when(i < _NTOT)
        def _():
            slot = i % _NBUF

            @pl.when(i >= _NBUF)
            def _():
                _out_copy(i - _NBUF, slot, "wait")

            _in_copy(i, slot, "start")

        @pl.when(i >= _LAG)
        def _():
            j = i - _LAG
            slot_j = j % _NBUF
            _in_copy(j, slot_j, "wait")
            _maybe_pool(j, slot_j)
            _out_copy(j, slot_j, "start")

        return carry

    jax.lax.fori_loop(0, _NTOT + _LAG, _loop, 0)

    for j in range(_NTOT - _NBUF, _NTOT):
        pltpu.make_async_copy(stage.at[j % _NBUF], io4.at[j - _NF1],
                              sem_out.at[j % _NBUF]).wait()


def _small_body(scal_ref, ts_ref, coords_ref, pooled_ref, patches_ref,
                f2b_ref, pb_ref, psb_ref, tb_ref, sfb_ref,
                f2o_ref, po_ref, pso_ref, to_ref, sfo_ref):
    r = pl.program_id(0)
    li = scal_ref[0]
    fn = scal_ref[1]

    @pl.when(r == li)
    def _():
        f2o_ref[0] = pooled_ref[0]
        po_ref[0] = patches_ref[0]
        xy = coords_ref[0]
        rp = xy[1:2, :] * ((_R_MAX - _R_MIN) / _FLS_H) + _R_MIN
        th = (xy[0:1, :] * (1.0 / _FLS_W) - 0.5) * (_FOV_H * jnp.pi / 180.0)
        pso_ref[0] = jnp.concatenate(
            [rp, th, jnp.zeros((1, _PPF), jnp.float32)], axis=0)
        sfo_ref[0] = jnp.full((1, _PPF), fn, dtype=jnp.int32)

    @pl.when(r != li)
    def _():
        f2o_ref[0] = f2b_ref[0]
        po_ref[0] = pb_ref[0]
        pso_ref[0] = psb_ref[0]
        sfo_ref[0] = sfb_ref[0]

    @pl.when(r == 0)
    def _():
        lanes = jax.lax.broadcasted_iota(jnp.int32, (1, _BUFF), 1)
        to_ref[...] = jnp.where(lanes == li, ts_ref[0, 0], tb_ref[...])


def kernel(fmap, imap, patches, coords, time_stamp, frame_n,
           fmap1_buf, fmap2_buf, imap_buf, patches_buf,
           patch_state_buf, time_buf, source_frame_buf):
    frame_n = jnp.asarray(frame_n, jnp.int32)
    li = frame_n % _BUFF
    scal = jnp.stack([li, frame_n])

    f32 = jnp.float32
    hbm = pl.BlockSpec(memory_space=pltpu.MemorySpace.HBM)
    smem = pl.BlockSpec(memory_space=pltpu.SMEM)
    vmem = pl.BlockSpec(memory_space=pltpu.VMEM)

    f4 = fmap.reshape(_CPR, _CH, _H, _W)
    i4 = imap.reshape(_CPR, _CH, _H, _W)
    f1b4 = fmap1_buf.reshape(_NF1, _CH, _H, _W)
    ib4 = imap_buf.reshape(_NF1, _CH, _H, _W)

    f1o4, io4, pooled = pl.pallas_call(
        _stream_body,
        in_specs=[smem, hbm, hbm, hbm, hbm],
        out_specs=[hbm, hbm, vmem],
        out_shape=[
            jax.ShapeDtypeStruct((_NF1, _CH, _H, _W), f32),
            jax.ShapeDtypeStruct((_NF1, _CH, _H, _W), f32),
            jax.ShapeDtypeStruct((_C, _PW, _PW), f32),
        ],
        scratch_shapes=[
            pltpu.VMEM((_NBUF, _CH, _H, _W), f32),
            pltpu.SemaphoreType.DMA((_NBUF,)),
            pltpu.SemaphoreType.DMA((_NBUF,)),
        ],
    )(scal, f4, i4, f1b4, ib4)

    fmap1_new = f1o4.reshape(_BUFF, _C, _H, _W)
    imap_new = io4.reshape(_BUFF, _C, _H, _W)

    pooled2 = pooled.reshape(1, _C, _PW * _PW)
    f2b2 = fmap2_buf.reshape(_BUFF, _C, _PW * _PW)
    pflat = patches.reshape(1, _PPF, _PK)
    pbflat = patches_buf.reshape(_BUFF, _PPF, _PK)
    coords2 = coords[0].T.reshape(1, 2, _PPF)
    ts2 = time_stamp.reshape(1, 1)
    ps3 = jnp.swapaxes(patch_state_buf, 1, 2)
    tb2 = time_buf.reshape(1, _BUFF)
    sf3 = source_frame_buf.reshape(_BUFF, 1, _PPF)

    small = pl.pallas_call(
        _small_body,
        grid_spec=pltpu.PrefetchScalarGridSpec(
            num_scalar_prefetch=1,
            grid=(_BUFF,),
            in_specs=[
                pl.BlockSpec((1, 1), lambda r, s: (0, 0)),
                pl.BlockSpec((1, 2, _PPF), lambda r, s: (0, 0, 0)),
                pl.BlockSpec((1, _C, _PW * _PW), lambda r, s: (0, 0, 0)),
                pl.BlockSpec((1, _PPF, _PK), lambda r, s: (0, 0, 0)),
                pl.BlockSpec((1, _C, _PW * _PW), lambda r, s: (r, 0, 0)),
                pl.BlockSpec((1, _PPF, _PK), lambda r, s: (r, 0, 0)),
                pl.BlockSpec((1, 3, _PPF), lambda r, s: (r, 0, 0)),
                pl.BlockSpec((1, _BUFF), lambda r, s: (0, 0)),
                pl.BlockSpec((1, 1, _PPF), lambda r, s: (r, 0, 0)),
            ],
            out_specs=[
                pl.BlockSpec((1, _C, _PW * _PW), lambda r, s: (r, 0, 0)),
                pl.BlockSpec((1, _PPF, _PK), lambda r, s: (r, 0, 0)),
                pl.BlockSpec((1, 3, _PPF), lambda r, s: (r, 0, 0)),
                pl.BlockSpec((1, _BUFF), lambda r, s: (0, 0)),
                pl.BlockSpec((1, 1, _PPF), lambda r, s: (r, 0, 0)),
            ],
        ),
        out_shape=[
            jax.ShapeDtypeStruct((_BUFF, _C, _PW * _PW), f32),
            jax.ShapeDtypeStruct((_BUFF, _PPF, _PK), f32),
            jax.ShapeDtypeStruct((_BUFF, 3, _PPF), f32),
            jax.ShapeDtypeStruct((1, _BUFF), f32),
            jax.ShapeDtypeStruct((_BUFF, 1, _PPF), jnp.int32),
        ],
    )
    f2new, pnew, psnew, tnew, sfnew = small(scal, ts2, coords2, pooled2,
                                            pflat, f2b2, pbflat, ps3,
                                            tb2, sf3)

    return (fmap1_new,
            f2new.reshape(_BUFF, _C, _PW, _PW),
            imap_new,
            pnew.reshape(_BUFF, _PPF, _C, _PATCH2),
            jnp.swapaxes(psnew, 1, 2),
            tnew.reshape(_BUFF),
            sfnew.reshape(_BUFF, _PPF))
